# baseline (speedup 1.0000x reference)
import functools

import jax
import jax.numpy as jnp
from jax import lax
from jax.experimental import pallas as pl
from jax.experimental.pallas import tpu as pltpu
from jax.experimental.pallas import tpu_sc as plsc

VOCAB = 832
EMB_DIM = 1024
D1 = 512
W_CNT = 848
NC = 2
NS = 16
NW = NC * NS
LANES = 16


def _sc_counts(x_flat, side):
    B = side.shape[0]
    b_per_w = B // NW
    CH = LANES
    n_ch = b_per_w // CH

    mesh = plsc.VectorSubcoreMesh(core_axis_name="c", subcore_axis_name="s")

    @functools.partial(
        pl.kernel,
        out_type=jax.ShapeDtypeStruct((B * W_CNT,), jnp.float32),
        mesh=mesh,
        compiler_params=pltpu.CompilerParams(needs_layout_passes=False),
        scratch_types=[
            pltpu.VMEM((CH * 64,), jnp.int32),
            pltpu.VMEM((CH,), jnp.float32),
            pltpu.VMEM((CH * W_CNT,), jnp.float32),
        ],
    )
    def k(x_hbm, side_hbm, out_hbm, idx_v, side_v, cnt_v):
        wid = lax.axis_index("s") * NC + lax.axis_index("c")
        row0w = wid * b_per_w
        lane = lax.iota(jnp.int32, 16)
        rowbase = lane * W_CNT
        colbase = lane * 64
        ones = jnp.ones((16,), jnp.float32)
        zeros = jnp.zeros((16,), jnp.float32)

        for i in range(CH * W_CNT // 16):
            cnt_v[pl.ds(i * 16, 16)] = zeros

        def chunk_body(ch, carry):
            row0 = row0w + ch * CH
            pltpu.sync_copy(x_hbm.at[pl.ds(row0 * 64, CH * 64)], idx_v)
            pltpu.sync_copy(side_hbm.at[pl.ds(row0, CH)], side_v)
            for sq in range(64):
                iv = plsc.load_gather(idx_v, [colbase + sq])
                plsc.addupdate_scatter(cnt_v, [rowbase + iv], ones)
            plsc.store_scatter(cnt_v, [rowbase + VOCAB], side_v[...])
            pltpu.sync_copy(cnt_v, out_hbm.at[pl.ds(row0 * W_CNT, CH * W_CNT)])
            for sq in range(64):
                iv = plsc.load_gather(idx_v, [colbase + sq])
                plsc.store_scatter(cnt_v, [rowbase + iv], zeros)
            return carry

        lax.fori_loop(0, n_ch, chunk_body, 0)

    return k(x_flat, side)


def _tc_fold(emb, w1a, w1s):
    def body(emb_ref, w1a_ref, w1s_ref, hi_ref, lo_ref):
        m = lax.dot_general(
            emb_ref[...], w1a_ref[...], (((1,), (1,)), ((), ())),
            preferred_element_type=jnp.float32)
        pad = jnp.zeros((W_CNT - VOCAB - 1, D1), jnp.float32)
        m = jnp.concatenate([m, w1s_ref[...], pad], axis=0)
        hi = m.astype(jnp.bfloat16)
        hi_ref[...] = hi
        lo_ref[...] = (m - hi.astype(jnp.float32)).astype(jnp.bfloat16)

    return pl.pallas_call(
        body,
        out_shape=[jax.ShapeDtypeStruct((W_CNT, D1), jnp.bfloat16),
                   jax.ShapeDtypeStruct((W_CNT, D1), jnp.bfloat16)],
    )(emb, w1a, w1s)


def _tc_mlp(cnts, M_hi, M_lo, b1, W2, b2, W3, b3):
    B = cnts.shape[0]
    BT = 512
    nb = B // BT

    def body(c_ref, mh_ref, ml_ref, b1_ref, w2_ref, b2_ref, w3_ref, b3_ref,
             o_ref):
        cb = c_ref[...].astype(jnp.bfloat16)
        g = (jnp.dot(cb, mh_ref[...], preferred_element_type=jnp.float32)
             + jnp.dot(cb, ml_ref[...], preferred_element_type=jnp.float32))
        h1 = jnp.maximum(g + b1_ref[...], 0.0)
        h2 = lax.dot_general(h1, w2_ref[...], (((1,), (1,)), ((), ())),
                             preferred_element_type=jnp.float32)
        h2 = jnp.maximum(h2 + b2_ref[...], 0.0)
        h3 = jnp.sum(h2 * w3_ref[...], axis=1, keepdims=True)
        o_ref[...] = h3 + b3_ref[0, 0]

    return pl.pallas_call(
        body,
        grid=(nb,),
        in_specs=[
            pl.BlockSpec((BT, W_CNT), lambda i: (i, 0)),
            pl.BlockSpec((W_CNT, D1), lambda i: (0, 0)),
            pl.BlockSpec((W_CNT, D1), lambda i: (0, 0)),
            pl.BlockSpec((1, D1), lambda i: (0, 0)),
            pl.BlockSpec((256, D1), lambda i: (0, 0)),
            pl.BlockSpec((1, 256), lambda i: (0, 0)),
            pl.BlockSpec((1, 256), lambda i: (0, 0)),
            pl.BlockSpec(memory_space=pltpu.SMEM),
        ],
        out_specs=pl.BlockSpec((BT, 1), lambda i: (i, 0)),
        out_shape=jax.ShapeDtypeStruct((B, 1), jnp.float32),
    )(cnts, M_hi, M_lo, b1, W2, b2, W3, b3)


def kernel(x, side_to_move, emb, W1, b1, W2, b2, W3, b3):
    B = x.shape[0]
    x_flat = x.astype(jnp.int32).reshape(B * 64)
    w1a = W1[:, :EMB_DIM]
    w1s = W1[:, EMB_DIM].reshape(1, D1)
    cnts = _sc_counts(x_flat, side_to_move).reshape(B, W_CNT)
    M_hi, M_lo = _tc_fold(emb, w1a, w1s)
    return _tc_mlp(cnts, M_hi, M_lo, b1.reshape(1, D1), W2,
                   b2.reshape(1, 256), W3, b3.reshape(1, 1))


# speedup vs baseline: 1.1512x; 1.1512x over previous
import functools

import jax
import jax.numpy as jnp
from jax import lax
from jax.experimental import pallas as pl
from jax.experimental.pallas import tpu as pltpu
from jax.experimental.pallas import tpu_sc as plsc

VOCAB = 832
EMB_DIM = 1024
D1 = 512
W_CNT = 848
NC = 2
NS = 16
NW = NC * NS
LANES = 16


def _sc_counts(x_flat, side):
    B = side.shape[0]
    b_per_w = B // NW
    CH = LANES
    n_ch = b_per_w // CH

    mesh = plsc.VectorSubcoreMesh(core_axis_name="c", subcore_axis_name="s")

    @functools.partial(
        pl.kernel,
        out_type=jax.ShapeDtypeStruct((B * W_CNT,), jnp.float32),
        mesh=mesh,
        compiler_params=pltpu.CompilerParams(needs_layout_passes=False),
        scratch_types=[
            pltpu.VMEM((CH * 64,), jnp.int32),
            pltpu.VMEM((CH * 64,), jnp.int32),
            pltpu.VMEM((CH,), jnp.float32),
            pltpu.VMEM((CH,), jnp.float32),
            pltpu.VMEM((CH * W_CNT,), jnp.float32),
            pltpu.VMEM((CH * W_CNT,), jnp.float32),
            pltpu.SemaphoreType.DMA,
            pltpu.SemaphoreType.DMA,
            pltpu.SemaphoreType.DMA,
        ],
    )
    def k(x_hbm, side_hbm, out_hbm, idx_v0, idx_v1, side_v0, side_v1,
          cnt_v0, cnt_v1, isem, ssem, osem):
        idx_b = (idx_v0, idx_v1)
        side_b = (side_v0, side_v1)
        cnt_b = (cnt_v0, cnt_v1)
        wid = lax.axis_index("s") * NC + lax.axis_index("c")
        row0w = wid * b_per_w
        lane = lax.iota(jnp.int32, 16)
        rowbase = lane * W_CNT
        colbase = lane * 64
        ones = jnp.ones((16,), jnp.float32)
        zeros = jnp.zeros((16,), jnp.float32)

        def fire_in(ch, b):
            row0 = row0w + ch * CH
            pltpu.async_copy(x_hbm.at[pl.ds(row0 * 64, CH * 64)],
                             idx_b[b], isem)
            pltpu.async_copy(side_hbm.at[pl.ds(row0, CH)], side_b[b], ssem)

        def zero_buf(b):
            for i in range(CH * W_CNT // 16):
                cnt_b[b][pl.ds(i * 16, 16)] = zeros

        for b in range(2):
            fire_in(b, b)
            zero_buf(b)

        def chunk_pair(g, carry):
            for b in range(2):
                ch = 2 * g + b
                row0 = row0w + ch * CH
                pltpu.make_async_copy(
                    x_hbm.at[pl.ds(row0 * 64, CH * 64)], idx_b[b],
                    isem).wait()
                pltpu.make_async_copy(
                    side_hbm.at[pl.ds(row0, CH)], side_b[b], ssem).wait()

                @pl.when(ch >= 2)
                def _drain():
                    pltpu.make_async_copy(
                        cnt_b[b],
                        out_hbm.at[pl.ds(row0 * W_CNT, CH * W_CNT)],
                        osem).wait()
                    zero_buf(b)

                for sq in range(64):
                    iv = plsc.load_gather(idx_b[b], [colbase + sq])
                    plsc.addupdate_scatter(cnt_b[b], [rowbase + iv], ones)
                plsc.store_scatter(cnt_b[b], [rowbase + VOCAB],
                                   side_b[b][...])
                pltpu.async_copy(
                    cnt_b[b],
                    out_hbm.at[pl.ds(row0 * W_CNT, CH * W_CNT)], osem)

                @pl.when(ch + 2 < n_ch)
                def _prefetch():
                    fire_in(ch + 2, b)
            return carry

        lax.fori_loop(0, n_ch // 2, chunk_pair, 0)

        for b in range(2):
            pltpu.make_async_copy(
                cnt_b[b], out_hbm.at[pl.ds(row0w * W_CNT, CH * W_CNT)],
                osem).wait()

    return k(x_flat, side)


def _tc_fold(emb, w1a, w1s):
    def body(emb_ref, w1a_ref, w1s_ref, hi_ref, lo_ref):
        m = lax.dot_general(
            emb_ref[...], w1a_ref[...], (((1,), (1,)), ((), ())),
            preferred_element_type=jnp.float32)
        pad = jnp.zeros((W_CNT - VOCAB - 1, D1), jnp.float32)
        m = jnp.concatenate([m, w1s_ref[...], pad], axis=0)
        hi = m.astype(jnp.bfloat16)
        hi_ref[...] = hi
        lo_ref[...] = (m - hi.astype(jnp.float32)).astype(jnp.bfloat16)

    return pl.pallas_call(
        body,
        out_shape=[jax.ShapeDtypeStruct((W_CNT, D1), jnp.bfloat16),
                   jax.ShapeDtypeStruct((W_CNT, D1), jnp.bfloat16)],
    )(emb, w1a, w1s)


def _tc_mlp(cnts, M_hi, M_lo, b1, W2, b2, W3, b3):
    B = cnts.shape[0]
    BT = 512
    nb = B // BT

    def body(c_ref, mh_ref, ml_ref, b1_ref, w2_ref, b2_ref, w3_ref, b3_ref,
             o_ref):
        cb = c_ref[...].astype(jnp.bfloat16)
        g = (jnp.dot(cb, mh_ref[...], preferred_element_type=jnp.float32)
             + jnp.dot(cb, ml_ref[...], preferred_element_type=jnp.float32))
        h1 = jnp.maximum(g + b1_ref[...], 0.0)
        h2 = lax.dot_general(h1, w2_ref[...], (((1,), (1,)), ((), ())),
                             preferred_element_type=jnp.float32)
        h2 = jnp.maximum(h2 + b2_ref[...], 0.0)
        h3 = jnp.sum(h2 * w3_ref[...], axis=1, keepdims=True)
        o_ref[...] = h3 + b3_ref[0, 0]

    return pl.pallas_call(
        body,
        grid=(nb,),
        in_specs=[
            pl.BlockSpec((BT, W_CNT), lambda i: (i, 0)),
            pl.BlockSpec((W_CNT, D1), lambda i: (0, 0)),
            pl.BlockSpec((W_CNT, D1), lambda i: (0, 0)),
            pl.BlockSpec((1, D1), lambda i: (0, 0)),
            pl.BlockSpec((256, D1), lambda i: (0, 0)),
            pl.BlockSpec((1, 256), lambda i: (0, 0)),
            pl.BlockSpec((1, 256), lambda i: (0, 0)),
            pl.BlockSpec(memory_space=pltpu.SMEM),
        ],
        out_specs=pl.BlockSpec((BT, 1), lambda i: (i, 0)),
        out_shape=jax.ShapeDtypeStruct((B, 1), jnp.float32),
    )(cnts, M_hi, M_lo, b1, W2, b2, W3, b3)


def kernel(x, side_to_move, emb, W1, b1, W2, b2, W3, b3):
    B = x.shape[0]
    x_flat = x.astype(jnp.int32).reshape(B * 64)
    w1a = W1[:, :EMB_DIM]
    w1s = W1[:, EMB_DIM].reshape(1, D1)
    cnts = _sc_counts(x_flat, side_to_move).reshape(B, W_CNT)
    M_hi, M_lo = _tc_fold(emb, w1a, w1s)
    return _tc_mlp(cnts, M_hi, M_lo, b1.reshape(1, D1), W2,
                   b2.reshape(1, 256), W3, b3.reshape(1, 1))


# speedup vs baseline: 1.5267x; 1.3262x over previous
import functools

import jax
import jax.numpy as jnp
from jax import lax
from jax.experimental import pallas as pl
from jax.experimental.pallas import tpu as pltpu
from jax.experimental.pallas import tpu_sc as plsc

VOCAB = 832
EMB_DIM = 1024
D1 = 512
W_CNT = 896
NC = 2
NS = 16
NW = NC * NS
LANES = 16


def _sc_counts(x_flat, side):
    B = side.shape[0]
    b_per_w = B // NW
    CH = LANES
    n_ch = b_per_w // CH

    mesh = plsc.VectorSubcoreMesh(core_axis_name="c", subcore_axis_name="s")

    @functools.partial(
        pl.kernel,
        out_type=jax.ShapeDtypeStruct((B, W_CNT), jnp.float32),
        mesh=mesh,
        compiler_params=pltpu.CompilerParams(needs_layout_passes=False),
        scratch_types=[
            pltpu.VMEM((CH * 64,), jnp.int32),
            pltpu.VMEM((CH * 64,), jnp.int32),
            pltpu.VMEM((CH,), jnp.float32),
            pltpu.VMEM((CH,), jnp.float32),
            pltpu.VMEM((CH, W_CNT), jnp.float32),
            pltpu.VMEM((CH, W_CNT), jnp.float32),
            pltpu.SemaphoreType.DMA,
            pltpu.SemaphoreType.DMA,
            pltpu.SemaphoreType.DMA,
        ],
    )
    def k(x_hbm, side_hbm, out_hbm, idx_v0, idx_v1, side_v0, side_v1,
          cnt_v0, cnt_v1, isem, ssem, osem):
        idx_b = (idx_v0, idx_v1)
        side_b = (side_v0, side_v1)
        cnt_b = (cnt_v0, cnt_v1)
        wid = lax.axis_index("s") * NC + lax.axis_index("c")
        row0w = wid * b_per_w
        lane = lax.iota(jnp.int32, 16)
        colbase = lane * 64
        ones = jnp.ones((16,), jnp.float32)
        zeros = jnp.zeros((16,), jnp.float32)

        def fire_in(ch, b):
            row0 = row0w + ch * CH
            pltpu.async_copy(x_hbm.at[pl.ds(row0 * 64, CH * 64)],
                             idx_b[b], isem)
            pltpu.async_copy(side_hbm.at[pl.ds(row0, CH)], side_b[b], ssem)

        def zero_buf(b):
            for r in range(CH):
                for i in range(W_CNT // 16):
                    cnt_b[b][r, pl.ds(i * 16, 16)] = zeros

        for b in range(2):
            fire_in(b, b)
            zero_buf(b)

        def chunk_pair(g, carry):
            for b in range(2):
                ch = 2 * g + b
                row0 = row0w + ch * CH
                pltpu.make_async_copy(
                    x_hbm.at[pl.ds(row0 * 64, CH * 64)], idx_b[b],
                    isem).wait()
                pltpu.make_async_copy(
                    side_hbm.at[pl.ds(row0, CH)], side_b[b], ssem).wait()

                @pl.when(ch >= 2)
                def _drain():
                    pltpu.make_async_copy(
                        cnt_b[b],
                        out_hbm.at[pl.ds(row0, CH)],
                        osem).wait()
                    zero_buf(b)

                for sq in range(64):
                    iv = plsc.load_gather(idx_b[b], [colbase + sq])
                    plsc.addupdate_scatter(cnt_b[b], [lane, iv], ones)
                plsc.store_scatter(cnt_b[b], [lane, jnp.full((16,), VOCAB, jnp.int32)],
                                   side_b[b][...])
                pltpu.async_copy(
                    cnt_b[b],
                    out_hbm.at[pl.ds(row0, CH)], osem)

                @pl.when(ch + 2 < n_ch)
                def _prefetch():
                    fire_in(ch + 2, b)
            return carry

        lax.fori_loop(0, n_ch // 2, chunk_pair, 0)

        for b in range(2):
            pltpu.make_async_copy(
                cnt_b[b], out_hbm.at[pl.ds(row0w, CH)], osem).wait()

    return k(x_flat, side)


def _tc_fold(emb, w1a, w1s):
    def body(emb_ref, w1a_ref, w1s_ref, hi_ref, lo_ref):
        m = lax.dot_general(
            emb_ref[...], w1a_ref[...], (((1,), (1,)), ((), ())),
            preferred_element_type=jnp.float32)
        pad = jnp.zeros((W_CNT - VOCAB - 1, D1), jnp.float32)
        m = jnp.concatenate([m, w1s_ref[...], pad], axis=0)
        hi = m.astype(jnp.bfloat16)
        hi_ref[...] = hi
        lo_ref[...] = (m - hi.astype(jnp.float32)).astype(jnp.bfloat16)

    return pl.pallas_call(
        body,
        out_shape=[jax.ShapeDtypeStruct((W_CNT, D1), jnp.bfloat16),
                   jax.ShapeDtypeStruct((W_CNT, D1), jnp.bfloat16)],
    )(emb, w1a, w1s)


def _tc_mlp(cnts, M_hi, M_lo, b1, W2, b2, W3, b3):
    B = cnts.shape[0]
    BT = 512
    nb = B // BT

    def body(c_ref, mh_ref, ml_ref, b1_ref, w2_ref, b2_ref, w3_ref, b3_ref,
             o_ref):
        cb = c_ref[...].astype(jnp.bfloat16)
        g = (jnp.dot(cb, mh_ref[...], preferred_element_type=jnp.float32)
             + jnp.dot(cb, ml_ref[...], preferred_element_type=jnp.float32))
        h1 = jnp.maximum(g + b1_ref[...], 0.0)
        h2 = lax.dot_general(h1, w2_ref[...], (((1,), (1,)), ((), ())),
                             preferred_element_type=jnp.float32)
        h2 = jnp.maximum(h2 + b2_ref[...], 0.0)
        h3 = jnp.sum(h2 * w3_ref[...], axis=1, keepdims=True)
        o_ref[...] = h3 + b3_ref[0, 0]

    return pl.pallas_call(
        body,
        grid=(nb,),
        in_specs=[
            pl.BlockSpec((BT, W_CNT), lambda i: (i, 0)),
            pl.BlockSpec((W_CNT, D1), lambda i: (0, 0)),
            pl.BlockSpec((W_CNT, D1), lambda i: (0, 0)),
            pl.BlockSpec((1, D1), lambda i: (0, 0)),
            pl.BlockSpec((256, D1), lambda i: (0, 0)),
            pl.BlockSpec((1, 256), lambda i: (0, 0)),
            pl.BlockSpec((1, 256), lambda i: (0, 0)),
            pl.BlockSpec(memory_space=pltpu.SMEM),
        ],
        out_specs=pl.BlockSpec((BT, 1), lambda i: (i, 0)),
        out_shape=jax.ShapeDtypeStruct((B, 1), jnp.float32),
    )(cnts, M_hi, M_lo, b1, W2, b2, W3, b3)


def kernel(x, side_to_move, emb, W1, b1, W2, b2, W3, b3):
    B = x.shape[0]
    x_flat = x.astype(jnp.int32).reshape(B * 64)
    w1a = W1[:, :EMB_DIM]
    w1s = W1[:, EMB_DIM].reshape(1, D1)
    cnts = _sc_counts(x_flat, side_to_move)
    M_hi, M_lo = _tc_fold(emb, w1a, w1s)
    return _tc_mlp(cnts, M_hi, M_lo, b1.reshape(1, D1), W2,
                   b2.reshape(1, 256), W3, b3.reshape(1, 1))


# speedup vs baseline: 1.5783x; 1.0339x over previous
import functools

import jax
import jax.numpy as jnp
from jax import lax
from jax.experimental import pallas as pl
from jax.experimental.pallas import tpu as pltpu
from jax.experimental.pallas import tpu_sc as plsc

VOCAB = 832
EMB_DIM = 1024
D1 = 512
W_CNT = 896
NC = 2
NS = 16
NW = NC * NS
LANES = 16


def _sc_counts(x_flat, side):
    B = side.shape[0]
    b_per_w = B // NW
    CH = LANES
    n_ch = b_per_w // CH

    mesh = plsc.VectorSubcoreMesh(core_axis_name="c", subcore_axis_name="s")

    @functools.partial(
        pl.kernel,
        out_type=jax.ShapeDtypeStruct((B, W_CNT), jnp.float32),
        mesh=mesh,
        compiler_params=pltpu.CompilerParams(needs_layout_passes=False),
        scratch_types=[
            pltpu.VMEM((CH * 64,), jnp.int32),
            pltpu.VMEM((CH * 64,), jnp.int32),
            pltpu.VMEM((CH * 64,), jnp.int32),
            pltpu.VMEM((CH * 64,), jnp.int32),
            pltpu.VMEM((CH,), jnp.float32),
            pltpu.VMEM((CH,), jnp.float32),
            pltpu.VMEM((CH, W_CNT), jnp.float32),
            pltpu.VMEM((CH, W_CNT), jnp.float32),
            pltpu.SemaphoreType.DMA,
            pltpu.SemaphoreType.DMA,
            pltpu.SemaphoreType.DMA,
        ],
    )
    def k(x_hbm, side_hbm, out_hbm, idx_v0, idx_v1, idx_v2, idx_v3,
          side_v0, side_v1, cnt_v0, cnt_v1, isem, ssem, osem):
        idx_b = (idx_v0, idx_v1, idx_v2, idx_v3)
        side_b = (side_v0, side_v1)
        cnt_b = (cnt_v0, cnt_v1)
        wid = lax.axis_index("s") * NC + lax.axis_index("c")
        row0w = wid * b_per_w
        lane = lax.iota(jnp.int32, 16)
        colbase = lane * 64
        ones = jnp.ones((16,), jnp.float32)
        zeros = jnp.zeros((16,), jnp.float32)

        def fire_in(ch, ib, sb):
            row0 = row0w + ch * CH
            pltpu.async_copy(x_hbm.at[pl.ds(row0 * 64, CH * 64)],
                             idx_b[ib], isem)
            pltpu.async_copy(side_hbm.at[pl.ds(row0, CH)], side_b[sb], ssem)

        def zero_buf(cb):
            for r in range(CH):
                for i in range(W_CNT // 16):
                    cnt_b[cb][r, pl.ds(i * 16, 16)] = zeros

        for b in range(2):
            fire_in(b, b, b)
            zero_buf(b)

        def chunk_quad(g, carry):
            for b in range(4):
                ch = 4 * g + b
                cb = b % 2
                row0 = row0w + ch * CH
                pltpu.make_async_copy(
                    x_hbm.at[pl.ds(row0 * 64, CH * 64)], idx_b[b],
                    isem).wait()
                pltpu.make_async_copy(
                    side_hbm.at[pl.ds(row0, CH)], side_b[cb], ssem).wait()

                @pl.when(ch >= 2)
                def _drain():
                    pltpu.make_async_copy(
                        cnt_b[cb],
                        out_hbm.at[pl.ds(row0, CH)],
                        osem).wait()
                    for sq in range(64):
                        iv = plsc.load_gather(idx_b[(b + 2) % 4],
                                              [colbase + sq])
                        plsc.store_scatter(cnt_b[cb], [lane, iv], zeros)

                for sq in range(64):
                    iv = plsc.load_gather(idx_b[b], [colbase + sq])
                    plsc.addupdate_scatter(cnt_b[cb], [lane, iv], ones)
                plsc.store_scatter(cnt_b[cb],
                                   [lane, jnp.full((16,), VOCAB, jnp.int32)],
                                   side_b[cb][...])
                pltpu.async_copy(
                    cnt_b[cb],
                    out_hbm.at[pl.ds(row0, CH)], osem)

                @pl.when(ch + 2 < n_ch)
                def _prefetch():
                    fire_in(ch + 2, (b + 2) % 4, cb)
            return carry

        lax.fori_loop(0, n_ch // 4, chunk_quad, 0)

        for b in range(2):
            pltpu.make_async_copy(
                cnt_b[b], out_hbm.at[pl.ds(row0w, CH)], osem).wait()

    return k(x_flat, side)


def _tc_fold(emb, w1a, w1s):
    def body(emb_ref, w1a_ref, w1s_ref, hi_ref, lo_ref):
        m = lax.dot_general(
            emb_ref[...], w1a_ref[...], (((1,), (1,)), ((), ())),
            preferred_element_type=jnp.float32)
        pad = jnp.zeros((W_CNT - VOCAB - 1, D1), jnp.float32)
        m = jnp.concatenate([m, w1s_ref[...], pad], axis=0)
        hi = m.astype(jnp.bfloat16)
        hi_ref[...] = hi
        lo_ref[...] = (m - hi.astype(jnp.float32)).astype(jnp.bfloat16)

    return pl.pallas_call(
        body,
        out_shape=[jax.ShapeDtypeStruct((W_CNT, D1), jnp.bfloat16),
                   jax.ShapeDtypeStruct((W_CNT, D1), jnp.bfloat16)],
    )(emb, w1a, w1s)


def _tc_mlp(cnts, M_hi, M_lo, b1, W2, b2, W3, b3):
    B = cnts.shape[0]
    BT = 512
    nb = B // BT

    def body(c_ref, mh_ref, ml_ref, b1_ref, w2_ref, b2_ref, w3_ref, b3_ref,
             o_ref):
        cb = c_ref[...].astype(jnp.bfloat16)
        g = (jnp.dot(cb, mh_ref[...], preferred_element_type=jnp.float32)
             + jnp.dot(cb, ml_ref[...], preferred_element_type=jnp.float32))
        h1 = jnp.maximum(g + b1_ref[...], 0.0)
        h2 = lax.dot_general(h1, w2_ref[...], (((1,), (1,)), ((), ())),
                             preferred_element_type=jnp.float32)
        h2 = jnp.maximum(h2 + b2_ref[...], 0.0)
        h3 = jnp.sum(h2 * w3_ref[...], axis=1, keepdims=True)
        o_ref[...] = h3 + b3_ref[0, 0]

    return pl.pallas_call(
        body,
        grid=(nb,),
        in_specs=[
            pl.BlockSpec((BT, W_CNT), lambda i: (i, 0)),
            pl.BlockSpec((W_CNT, D1), lambda i: (0, 0)),
            pl.BlockSpec((W_CNT, D1), lambda i: (0, 0)),
            pl.BlockSpec((1, D1), lambda i: (0, 0)),
            pl.BlockSpec((256, D1), lambda i: (0, 0)),
            pl.BlockSpec((1, 256), lambda i: (0, 0)),
            pl.BlockSpec((1, 256), lambda i: (0, 0)),
            pl.BlockSpec(memory_space=pltpu.SMEM),
        ],
        out_specs=pl.BlockSpec((BT, 1), lambda i: (i, 0)),
        out_shape=jax.ShapeDtypeStruct((B, 1), jnp.float32),
    )(cnts, M_hi, M_lo, b1, W2, b2, W3, b3)


def kernel(x, side_to_move, emb, W1, b1, W2, b2, W3, b3):
    B = x.shape[0]
    x_flat = x.astype(jnp.int32).reshape(B * 64)
    w1a = W1[:, :EMB_DIM]
    w1s = W1[:, EMB_DIM].reshape(1, D1)
    cnts = _sc_counts(x_flat, side_to_move)
    M_hi, M_lo = _tc_fold(emb, w1a, w1s)
    return _tc_mlp(cnts, M_hi, M_lo, b1.reshape(1, D1), W2,
                   b2.reshape(1, 256), W3, b3.reshape(1, 1))


# speedup vs baseline: 1.7013x; 1.0779x over previous
import functools

import jax
import jax.numpy as jnp
from jax import lax
from jax.experimental import pallas as pl
from jax.experimental.pallas import tpu as pltpu
from jax.experimental.pallas import tpu_sc as plsc

VOCAB = 832
EMB_DIM = 1024
D1 = 512
W_CNT = 896
NC = 2
NS = 16
NW = NC * NS
LANES = 16


def _sc_counts(x2d, side):
    B = side.shape[0]
    b_per_w = B // NW
    CH = LANES
    n_ch = b_per_w // CH

    mesh = plsc.VectorSubcoreMesh(core_axis_name="c", subcore_axis_name="s")

    @functools.partial(
        pl.kernel,
        out_type=jax.ShapeDtypeStruct((B, W_CNT), jnp.float32),
        mesh=mesh,
        compiler_params=pltpu.CompilerParams(needs_layout_passes=False),
        scratch_types=[
            pltpu.VMEM((CH, 64), jnp.int32),
            pltpu.VMEM((CH, 64), jnp.int32),
            pltpu.VMEM((CH, 64), jnp.int32),
            pltpu.VMEM((CH, 64), jnp.int32),
            pltpu.VMEM((CH,), jnp.float32),
            pltpu.VMEM((CH,), jnp.float32),
            pltpu.VMEM((CH, W_CNT), jnp.float32),
            pltpu.VMEM((CH, W_CNT), jnp.float32),
            pltpu.SemaphoreType.DMA,
            pltpu.SemaphoreType.DMA,
            pltpu.SemaphoreType.DMA,
        ],
    )
    def k(x_hbm, side_hbm, out_hbm, idx_v0, idx_v1, idx_v2, idx_v3,
          side_v0, side_v1, cnt_v0, cnt_v1, isem, ssem, osem):
        idx_b = (idx_v0, idx_v1, idx_v2, idx_v3)
        side_b = (side_v0, side_v1)
        cnt_b = (cnt_v0, cnt_v1)
        wid = lax.axis_index("s") * NC + lax.axis_index("c")
        row0w = wid * b_per_w
        lane = lax.iota(jnp.int32, 16)
        ones = jnp.ones((16,), jnp.float32)
        zeros = jnp.zeros((16,), jnp.float32)

        def fire_in(ch, ib, sb):
            row0 = row0w + ch * CH
            pltpu.async_copy(x_hbm.at[pl.ds(row0, CH)], idx_b[ib], isem)
            pltpu.async_copy(side_hbm.at[pl.ds(row0, CH)], side_b[sb], ssem)

        def zero_buf(cb):
            for r in range(CH):
                for i in range(W_CNT // 16):
                    cnt_b[cb][r, pl.ds(i * 16, 16)] = zeros

        for b in range(2):
            fire_in(b, b, b)
            zero_buf(b)

        def chunk_quad(g, carry):
            for b in range(4):
                ch = 4 * g + b
                cb = b % 2
                row0 = row0w + ch * CH
                pltpu.make_async_copy(
                    x_hbm.at[pl.ds(row0, CH)], idx_b[b], isem).wait()
                pltpu.make_async_copy(
                    side_hbm.at[pl.ds(row0, CH)], side_b[cb], ssem).wait()

                @pl.when(ch >= 2)
                def _drain():
                    pltpu.make_async_copy(
                        cnt_b[cb],
                        out_hbm.at[pl.ds(row0, CH)],
                        osem).wait()
                    for sq in range(64):
                        iv = plsc.load_gather(
                            idx_b[(b + 2) % 4],
                            [lane, jnp.full((16,), sq, jnp.int32)])
                        plsc.store_scatter(cnt_b[cb], [lane, iv], zeros)

                for sq in range(64):
                    iv = plsc.load_gather(
                        idx_b[b], [lane, jnp.full((16,), sq, jnp.int32)])
                    plsc.addupdate_scatter(cnt_b[cb], [lane, iv], ones)
                plsc.store_scatter(cnt_b[cb],
                                   [lane, jnp.full((16,), VOCAB, jnp.int32)],
                                   side_b[cb][...])
                pltpu.async_copy(
                    cnt_b[cb],
                    out_hbm.at[pl.ds(row0, CH)], osem)

                @pl.when(ch + 2 < n_ch)
                def _prefetch():
                    fire_in(ch + 2, (b + 2) % 4, cb)
            return carry

        lax.fori_loop(0, n_ch // 4, chunk_quad, 0)

        for b in range(2):
            pltpu.make_async_copy(
                cnt_b[b], out_hbm.at[pl.ds(row0w, CH)], osem).wait()

    return k(x2d, side)


def _tc_fold(emb, w1a, w1s):
    def body(emb_ref, w1a_ref, w1s_ref, hi_ref, lo_ref):
        m = lax.dot_general(
            emb_ref[...], w1a_ref[...], (((1,), (1,)), ((), ())),
            preferred_element_type=jnp.float32)
        pad = jnp.zeros((W_CNT - VOCAB - 1, D1), jnp.float32)
        m = jnp.concatenate([m, w1s_ref[...], pad], axis=0)
        hi = m.astype(jnp.bfloat16)
        hi_ref[...] = hi
        lo_ref[...] = (m - hi.astype(jnp.float32)).astype(jnp.bfloat16)

    return pl.pallas_call(
        body,
        out_shape=[jax.ShapeDtypeStruct((W_CNT, D1), jnp.bfloat16),
                   jax.ShapeDtypeStruct((W_CNT, D1), jnp.bfloat16)],
    )(emb, w1a, w1s)


def _tc_mlp(cnts, M_hi, M_lo, b1, W2, b2, W3, b3):
    B = cnts.shape[0]
    BT = 1024
    nb = B // BT

    def body(c_ref, mh_ref, ml_ref, b1_ref, w2_ref, b2_ref, w3_ref, b3_ref,
             o_ref):
        cb = c_ref[...].astype(jnp.bfloat16)
        g = (jnp.dot(cb, mh_ref[...], preferred_element_type=jnp.float32)
             + jnp.dot(cb, ml_ref[...], preferred_element_type=jnp.float32))
        h1 = jnp.maximum(g + b1_ref[...], 0.0)
        h2 = lax.dot_general(h1, w2_ref[...], (((1,), (1,)), ((), ())),
                             preferred_element_type=jnp.float32)
        h2 = jnp.maximum(h2 + b2_ref[...], 0.0)
        h3 = jnp.sum(h2 * w3_ref[...], axis=1, keepdims=True)
        o_ref[...] = h3 + b3_ref[0, 0]

    return pl.pallas_call(
        body,
        grid=(nb,),
        in_specs=[
            pl.BlockSpec((BT, W_CNT), lambda i: (i, 0)),
            pl.BlockSpec((W_CNT, D1), lambda i: (0, 0)),
            pl.BlockSpec((W_CNT, D1), lambda i: (0, 0)),
            pl.BlockSpec((1, D1), lambda i: (0, 0)),
            pl.BlockSpec((256, D1), lambda i: (0, 0)),
            pl.BlockSpec((1, 256), lambda i: (0, 0)),
            pl.BlockSpec((1, 256), lambda i: (0, 0)),
            pl.BlockSpec(memory_space=pltpu.SMEM),
        ],
        out_specs=pl.BlockSpec((BT, 1), lambda i: (i, 0)),
        out_shape=jax.ShapeDtypeStruct((B, 1), jnp.float32),
    )(cnts, M_hi, M_lo, b1, W2, b2, W3, b3)


def kernel(x, side_to_move, emb, W1, b1, W2, b2, W3, b3):
    B = x.shape[0]
    x2d = x.astype(jnp.int32)
    w1a = W1[:, :EMB_DIM]
    w1s = W1[:, EMB_DIM].reshape(1, D1)
    cnts = _sc_counts(x2d, side_to_move)
    M_hi, M_lo = _tc_fold(emb, w1a, w1s)
    return _tc_mlp(cnts, M_hi, M_lo, b1.reshape(1, D1), W2,
                   b2.reshape(1, 256), W3, b3.reshape(1, 1))


# speedup vs baseline: 1.7082x; 1.0040x over previous
import functools

import jax
import jax.numpy as jnp
from jax import lax
from jax.experimental import pallas as pl
from jax.experimental.pallas import tpu as pltpu
from jax.experimental.pallas import tpu_sc as plsc

VOCAB = 832
EMB_DIM = 1024
D1 = 512
W_CNT = 896
NC = 2
NS = 16
NW = NC * NS
LANES = 16


def _sc_counts(x2d, side):
    B = side.shape[0]
    b_per_w = B // NW
    CH = LANES
    n_ch = b_per_w // CH

    mesh = plsc.VectorSubcoreMesh(core_axis_name="c", subcore_axis_name="s")

    @functools.partial(
        pl.kernel,
        out_type=jax.ShapeDtypeStruct((B, W_CNT), jnp.float32),
        mesh=mesh,
        compiler_params=pltpu.CompilerParams(needs_layout_passes=False),
        scratch_types=[
            pltpu.VMEM((b_per_w, 64), jnp.int32),
            pltpu.VMEM((b_per_w,), jnp.float32),
            pltpu.VMEM((CH, W_CNT), jnp.float32),
            pltpu.VMEM((CH, W_CNT), jnp.float32),
            pltpu.VMEM((CH, W_CNT), jnp.float32),
            pltpu.VMEM((CH, W_CNT), jnp.float32),
            pltpu.SemaphoreType.DMA,
            pltpu.SemaphoreType.DMA,
        ],
    )
    def k(x_hbm, side_hbm, out_hbm, idx_all, side_all,
          cnt_v0, cnt_v1, cnt_v2, cnt_v3, isem, osem):
        cnt_b = (cnt_v0, cnt_v1, cnt_v2, cnt_v3)
        nbuf = len(cnt_b)
        wid = lax.axis_index("s") * NC + lax.axis_index("c")
        row0w = wid * b_per_w
        lane = lax.iota(jnp.int32, 16)
        ones = jnp.ones((16,), jnp.float32)
        zeros = jnp.zeros((16,), jnp.float32)

        pltpu.async_copy(x_hbm.at[pl.ds(row0w, b_per_w)], idx_all, isem)
        pltpu.async_copy(side_hbm.at[pl.ds(row0w, b_per_w)], side_all, isem)
        for cb in range(nbuf):
            for r in range(CH):
                for i in range(W_CNT // 16):
                    cnt_b[cb][r, pl.ds(i * 16, 16)] = zeros
        pltpu.make_async_copy(
            x_hbm.at[pl.ds(row0w, b_per_w)], idx_all, isem).wait()
        pltpu.make_async_copy(
            side_hbm.at[pl.ds(row0w, b_per_w)], side_all, isem).wait()

        def chunk_quad(g, carry):
            for cb in range(nbuf):
                ch = nbuf * g + cb
                row0 = row0w + ch * CH
                rows = ch * CH + lane

                @pl.when(ch >= nbuf)
                def _drain():
                    pltpu.make_async_copy(
                        cnt_b[cb],
                        out_hbm.at[pl.ds(row0, CH)],
                        osem).wait()
                    prev_rows = rows - nbuf * CH
                    for sq in range(64):
                        iv = plsc.load_gather(
                            idx_all,
                            [prev_rows, jnp.full((16,), sq, jnp.int32)])
                        plsc.store_scatter(cnt_b[cb], [lane, iv], zeros)

                for sq in range(64):
                    iv = plsc.load_gather(
                        idx_all, [rows, jnp.full((16,), sq, jnp.int32)])
                    plsc.addupdate_scatter(cnt_b[cb], [lane, iv], ones)
                plsc.store_scatter(cnt_b[cb],
                                   [lane, jnp.full((16,), VOCAB, jnp.int32)],
                                   side_all[pl.ds(ch * CH, CH)])
                pltpu.async_copy(
                    cnt_b[cb],
                    out_hbm.at[pl.ds(row0, CH)], osem)
            return carry

        lax.fori_loop(0, n_ch // nbuf, chunk_quad, 0)

        for cb in range(nbuf):
            pltpu.make_async_copy(
                cnt_b[cb], out_hbm.at[pl.ds(row0w, CH)], osem).wait()

    return k(x2d, side)


def _tc_fold(emb, w1a, w1s):
    def body(emb_ref, w1a_ref, w1s_ref, hi_ref, lo_ref):
        m = lax.dot_general(
            emb_ref[...], w1a_ref[...], (((1,), (1,)), ((), ())),
            preferred_element_type=jnp.float32)
        pad = jnp.zeros((W_CNT - VOCAB - 1, D1), jnp.float32)
        m = jnp.concatenate([m, w1s_ref[...], pad], axis=0)
        hi = m.astype(jnp.bfloat16)
        hi_ref[...] = hi
        lo_ref[...] = (m - hi.astype(jnp.float32)).astype(jnp.bfloat16)

    return pl.pallas_call(
        body,
        out_shape=[jax.ShapeDtypeStruct((W_CNT, D1), jnp.bfloat16),
                   jax.ShapeDtypeStruct((W_CNT, D1), jnp.bfloat16)],
    )(emb, w1a, w1s)


def _tc_mlp(cnts, M_hi, M_lo, b1, W2, b2, W3, b3):
    B = cnts.shape[0]
    BT = 1024
    nb = B // BT

    def body(c_ref, mh_ref, ml_ref, b1_ref, w2_ref, b2_ref, w3_ref, b3_ref,
             o_ref):
        cb = c_ref[...].astype(jnp.bfloat16)
        g = (jnp.dot(cb, mh_ref[...], preferred_element_type=jnp.float32)
             + jnp.dot(cb, ml_ref[...], preferred_element_type=jnp.float32))
        h1 = jnp.maximum(g + b1_ref[...], 0.0)
        h2 = lax.dot_general(h1, w2_ref[...], (((1,), (1,)), ((), ())),
                             preferred_element_type=jnp.float32)
        h2 = jnp.maximum(h2 + b2_ref[...], 0.0)
        h3 = jnp.sum(h2 * w3_ref[...], axis=1, keepdims=True)
        o_ref[...] = h3 + b3_ref[0, 0]

    return pl.pallas_call(
        body,
        grid=(nb,),
        in_specs=[
            pl.BlockSpec((BT, W_CNT), lambda i: (i, 0)),
            pl.BlockSpec((W_CNT, D1), lambda i: (0, 0)),
            pl.BlockSpec((W_CNT, D1), lambda i: (0, 0)),
            pl.BlockSpec((1, D1), lambda i: (0, 0)),
            pl.BlockSpec((256, D1), lambda i: (0, 0)),
            pl.BlockSpec((1, 256), lambda i: (0, 0)),
            pl.BlockSpec((1, 256), lambda i: (0, 0)),
            pl.BlockSpec(memory_space=pltpu.SMEM),
        ],
        out_specs=pl.BlockSpec((BT, 1), lambda i: (i, 0)),
        out_shape=jax.ShapeDtypeStruct((B, 1), jnp.float32),
    )(cnts, M_hi, M_lo, b1, W2, b2, W3, b3)


def kernel(x, side_to_move, emb, W1, b1, W2, b2, W3, b3):
    B = x.shape[0]
    x2d = x.astype(jnp.int32)
    w1a = W1[:, :EMB_DIM]
    w1s = W1[:, EMB_DIM].reshape(1, D1)
    cnts = _sc_counts(x2d, side_to_move)
    M_hi, M_lo = _tc_fold(emb, w1a, w1s)
    return _tc_mlp(cnts, M_hi, M_lo, b1.reshape(1, D1), W2,
                   b2.reshape(1, 256), W3, b3.reshape(1, 1))


# speedup vs baseline: 1.7105x; 1.0014x over previous
import functools

import jax
import jax.numpy as jnp
from jax import lax
from jax.experimental import pallas as pl
from jax.experimental.pallas import tpu as pltpu
from jax.experimental.pallas import tpu_sc as plsc

VOCAB = 832
EMB_DIM = 1024
D1 = 512
W_CNT = 896
NC = 2
NS = 16
NW = NC * NS
LANES = 16


def _sc_counts(x2d, side):
    B = side.shape[0]
    b_per_w = B // NW
    CH = LANES
    n_ch = b_per_w // CH

    mesh = plsc.VectorSubcoreMesh(core_axis_name="c", subcore_axis_name="s")

    @functools.partial(
        pl.kernel,
        out_type=jax.ShapeDtypeStruct((B, W_CNT), jnp.float32),
        mesh=mesh,
        compiler_params=pltpu.CompilerParams(needs_layout_passes=False),
        scratch_types=[
            pltpu.VMEM((b_per_w, 64), jnp.int32),
            pltpu.VMEM((b_per_w,), jnp.float32),
            pltpu.VMEM((CH, W_CNT), jnp.float32),
            pltpu.VMEM((CH, W_CNT), jnp.float32),
            pltpu.VMEM((CH, W_CNT), jnp.float32),
            pltpu.VMEM((CH, W_CNT), jnp.float32),
            pltpu.SemaphoreType.DMA,
            pltpu.SemaphoreType.DMA,
        ],
    )
    def k(x_hbm, side_hbm, out_hbm, idx_all, side_all,
          cnt_v0, cnt_v1, cnt_v2, cnt_v3, isem, osem):
        cnt_b = (cnt_v0, cnt_v1, cnt_v2, cnt_v3)
        nbuf = len(cnt_b)
        wid = lax.axis_index("s") * NC + lax.axis_index("c")
        row0w = wid * b_per_w
        lane = lax.iota(jnp.int32, 16)
        ones = jnp.ones((16,), jnp.float32)
        zeros = jnp.zeros((16,), jnp.float32)

        pltpu.async_copy(x_hbm.at[pl.ds(row0w, b_per_w)], idx_all, isem)
        pltpu.async_copy(side_hbm.at[pl.ds(row0w, b_per_w)], side_all, isem)
        for cb in range(nbuf):
            for r in range(CH):
                for i in range(W_CNT // 16):
                    cnt_b[cb][r, pl.ds(i * 16, 16)] = zeros
        pltpu.make_async_copy(
            x_hbm.at[pl.ds(row0w, b_per_w)], idx_all, isem).wait()
        pltpu.make_async_copy(
            side_hbm.at[pl.ds(row0w, b_per_w)], side_all, isem).wait()

        def chunk_quad(g, carry):
            for cb in range(nbuf):
                ch = nbuf * g + cb
                row0 = row0w + ch * CH
                rows = ch * CH + lane

                @pl.when(ch >= nbuf)
                def _drain():
                    pltpu.make_async_copy(
                        cnt_b[cb],
                        out_hbm.at[pl.ds(row0, CH)],
                        osem).wait()
                    prev_rows = rows - nbuf * CH
                    for sq in range(64):
                        iv = plsc.load_gather(
                            idx_all,
                            [prev_rows, jnp.full((16,), sq, jnp.int32)])
                        plsc.store_scatter(cnt_b[cb], [lane, iv], zeros)

                for sq in range(64):
                    iv = plsc.load_gather(
                        idx_all, [rows, jnp.full((16,), sq, jnp.int32)])
                    plsc.addupdate_scatter(cnt_b[cb], [lane, iv], ones)
                plsc.store_scatter(cnt_b[cb],
                                   [lane, jnp.full((16,), VOCAB, jnp.int32)],
                                   side_all[pl.ds(ch * CH, CH)])
                pltpu.async_copy(
                    cnt_b[cb],
                    out_hbm.at[pl.ds(row0, CH)], osem)
            return carry

        lax.fori_loop(0, n_ch // nbuf, chunk_quad, 0)

        for cb in range(nbuf):
            pltpu.make_async_copy(
                cnt_b[cb], out_hbm.at[pl.ds(row0w, CH)], osem).wait()

    return k(x2d, side)


def _tc_fold(emb, W1):
    def body(emb_ref, w1_ref, hi_ref, lo_ref):
        w1a = w1_ref[:, pl.ds(0, EMB_DIM)]
        m = lax.dot_general(
            emb_ref[...], w1a, (((1,), (1,)), ((), ())),
            preferred_element_type=jnp.float32,
            precision=lax.Precision.HIGHEST)
        w1s = jnp.reshape(w1_ref[:, pl.ds(EMB_DIM, 1)], (1, D1))
        pad = jnp.zeros((W_CNT - VOCAB - 1, D1), jnp.float32)
        m = jnp.concatenate([m, w1s, pad], axis=0)
        hi = m.astype(jnp.bfloat16)
        hi_ref[...] = hi
        lo_ref[...] = (m - hi.astype(jnp.float32)).astype(jnp.bfloat16)

    return pl.pallas_call(
        body,
        out_shape=[jax.ShapeDtypeStruct((W_CNT, D1), jnp.bfloat16),
                   jax.ShapeDtypeStruct((W_CNT, D1), jnp.bfloat16)],
    )(emb, W1)


def _tc_mlp(cnts, M_hi, M_lo, b1, W2, b2, W3, b3):
    B = cnts.shape[0]
    BT = 1024
    nb = B // BT

    def body(c_ref, mh_ref, ml_ref, b1_ref, w2_ref, b2_ref, w3_ref, b3_ref,
             o_ref):
        cb = c_ref[...].astype(jnp.bfloat16)
        g = (jnp.dot(cb, mh_ref[...], preferred_element_type=jnp.float32)
             + jnp.dot(cb, ml_ref[...], preferred_element_type=jnp.float32))
        h1 = jnp.maximum(g + b1_ref[...], 0.0)
        h2 = lax.dot_general(h1, w2_ref[...], (((1,), (1,)), ((), ())),
                             preferred_element_type=jnp.float32)
        h2 = jnp.maximum(h2 + b2_ref[...], 0.0)
        h3 = jnp.sum(h2 * w3_ref[...], axis=1, keepdims=True)
        o_ref[...] = h3 + b3_ref[0, 0]

    return pl.pallas_call(
        body,
        grid=(nb,),
        in_specs=[
            pl.BlockSpec((BT, W_CNT), lambda i: (i, 0)),
            pl.BlockSpec((W_CNT, D1), lambda i: (0, 0)),
            pl.BlockSpec((W_CNT, D1), lambda i: (0, 0)),
            pl.BlockSpec((1, D1), lambda i: (0, 0)),
            pl.BlockSpec((256, D1), lambda i: (0, 0)),
            pl.BlockSpec((1, 256), lambda i: (0, 0)),
            pl.BlockSpec((1, 256), lambda i: (0, 0)),
            pl.BlockSpec(memory_space=pltpu.SMEM),
        ],
        out_specs=pl.BlockSpec((BT, 1), lambda i: (i, 0)),
        out_shape=jax.ShapeDtypeStruct((B, 1), jnp.float32),
    )(cnts, M_hi, M_lo, b1, W2, b2, W3, b3)


def kernel(x, side_to_move, emb, W1, b1, W2, b2, W3, b3):
    B = x.shape[0]
    x2d = x.astype(jnp.int32)
    cnts = _sc_counts(x2d, side_to_move)
    M_hi, M_lo = _tc_fold(emb, W1)
    return _tc_mlp(cnts, M_hi, M_lo, b1.reshape(1, D1), W2,
                   b2.reshape(1, 256), W3, b3.reshape(1, 1))


# speedup vs baseline: 1.8300x; 1.0699x over previous
import functools

import jax
import jax.numpy as jnp
from jax import lax
from jax.experimental import pallas as pl
from jax.experimental.pallas import tpu as pltpu
from jax.experimental.pallas import tpu_sc as plsc

VOCAB = 832
EMB_DIM = 1024
D1 = 512
W_CNT = 896
NC = 2
NS = 16
NW = NC * NS
LANES = 16


def _sc_counts(x2d, side, base, rows):
    b_per_w = rows // NW
    CH = LANES
    n_ch = b_per_w // CH

    mesh = plsc.VectorSubcoreMesh(core_axis_name="c", subcore_axis_name="s")

    @functools.partial(
        pl.kernel,
        out_type=jax.ShapeDtypeStruct((rows, W_CNT), jnp.float32),
        mesh=mesh,
        compiler_params=pltpu.CompilerParams(needs_layout_passes=False),
        scratch_types=[
            pltpu.VMEM((b_per_w, 64), jnp.int32),
            pltpu.VMEM((b_per_w,), jnp.float32),
            pltpu.VMEM((CH, W_CNT), jnp.float32),
            pltpu.VMEM((CH, W_CNT), jnp.float32),
            pltpu.VMEM((CH, W_CNT), jnp.float32),
            pltpu.VMEM((CH, W_CNT), jnp.float32),
            pltpu.SemaphoreType.DMA,
            pltpu.SemaphoreType.DMA,
        ],
    )
    def k(x_hbm, side_hbm, out_hbm, idx_all, side_all,
          cnt_v0, cnt_v1, cnt_v2, cnt_v3, isem, osem):
        cnt_b = (cnt_v0, cnt_v1, cnt_v2, cnt_v3)
        nbuf = len(cnt_b)
        wid = lax.axis_index("s") * NC + lax.axis_index("c")
        row0w = wid * b_per_w
        in0w = base + row0w
        lane = lax.iota(jnp.int32, 16)
        ones = jnp.ones((16,), jnp.float32)
        zeros = jnp.zeros((16,), jnp.float32)

        pltpu.async_copy(x_hbm.at[pl.ds(in0w, b_per_w)], idx_all, isem)
        pltpu.async_copy(side_hbm.at[pl.ds(in0w, b_per_w)], side_all, isem)
        for cb in range(nbuf):
            for r in range(CH):
                for i in range(W_CNT // 16):
                    cnt_b[cb][r, pl.ds(i * 16, 16)] = zeros
        pltpu.make_async_copy(
            x_hbm.at[pl.ds(in0w, b_per_w)], idx_all, isem).wait()
        pltpu.make_async_copy(
            side_hbm.at[pl.ds(in0w, b_per_w)], side_all, isem).wait()

        def chunk_quad(g, carry):
            for cb in range(nbuf):
                ch = nbuf * g + cb
                row0 = row0w + ch * CH
                rows = ch * CH + lane

                @pl.when(ch >= nbuf)
                def _drain():
                    pltpu.make_async_copy(
                        cnt_b[cb],
                        out_hbm.at[pl.ds(row0, CH)],
                        osem).wait()
                    prev_rows = rows - nbuf * CH
                    for sq in range(64):
                        iv = plsc.load_gather(
                            idx_all,
                            [prev_rows, jnp.full((16,), sq, jnp.int32)])
                        plsc.store_scatter(cnt_b[cb], [lane, iv], zeros)

                for sq in range(64):
                    iv = plsc.load_gather(
                        idx_all, [rows, jnp.full((16,), sq, jnp.int32)])
                    plsc.addupdate_scatter(cnt_b[cb], [lane, iv], ones)
                plsc.store_scatter(cnt_b[cb],
                                   [lane, jnp.full((16,), VOCAB, jnp.int32)],
                                   side_all[pl.ds(ch * CH, CH)])
                pltpu.async_copy(
                    cnt_b[cb],
                    out_hbm.at[pl.ds(row0, CH)], osem)
            return carry

        lax.fori_loop(0, n_ch // nbuf, chunk_quad, 0)

        for cb in range(nbuf):
            pltpu.make_async_copy(
                cnt_b[cb], out_hbm.at[pl.ds(row0w, CH)], osem).wait()

    return k(x2d, side)


def _tc_fold(emb, W1):
    def body(emb_ref, w1_ref, hi_ref, lo_ref):
        w1a = w1_ref[:, pl.ds(0, EMB_DIM)]
        m = lax.dot_general(
            emb_ref[...], w1a, (((1,), (1,)), ((), ())),
            preferred_element_type=jnp.float32,
            precision=lax.Precision.HIGHEST)
        w1s = jnp.reshape(w1_ref[:, pl.ds(EMB_DIM, 1)], (1, D1))
        pad = jnp.zeros((W_CNT - VOCAB - 1, D1), jnp.float32)
        m = jnp.concatenate([m, w1s, pad], axis=0)
        hi = m.astype(jnp.bfloat16)
        hi_ref[...] = hi
        lo_ref[...] = (m - hi.astype(jnp.float32)).astype(jnp.bfloat16)

    return pl.pallas_call(
        body,
        out_shape=[jax.ShapeDtypeStruct((W_CNT, D1), jnp.bfloat16),
                   jax.ShapeDtypeStruct((W_CNT, D1), jnp.bfloat16)],
    )(emb, W1)


def _tc_mlp(cnts, M_hi, M_lo, b1, W2, b2, W3, b3):
    B = cnts.shape[0]
    BT = 1024
    nb = B // BT

    def body(c_ref, mh_ref, ml_ref, b1_ref, w2_ref, b2_ref, w3_ref, b3_ref,
             o_ref):
        cb = c_ref[...].astype(jnp.bfloat16)
        g = (jnp.dot(cb, mh_ref[...], preferred_element_type=jnp.float32)
             + jnp.dot(cb, ml_ref[...], preferred_element_type=jnp.float32))
        h1 = jnp.maximum(g + b1_ref[...], 0.0)
        h2 = lax.dot_general(h1, w2_ref[...], (((1,), (1,)), ((), ())),
                             preferred_element_type=jnp.float32)
        h2 = jnp.maximum(h2 + b2_ref[...], 0.0)
        h3 = jnp.sum(h2 * w3_ref[...], axis=1, keepdims=True)
        o_ref[...] = h3 + b3_ref[0, 0]

    return pl.pallas_call(
        body,
        grid=(nb,),
        in_specs=[
            pl.BlockSpec((BT, W_CNT), lambda i: (i, 0)),
            pl.BlockSpec((W_CNT, D1), lambda i: (0, 0)),
            pl.BlockSpec((W_CNT, D1), lambda i: (0, 0)),
            pl.BlockSpec((1, D1), lambda i: (0, 0)),
            pl.BlockSpec((256, D1), lambda i: (0, 0)),
            pl.BlockSpec((1, 256), lambda i: (0, 0)),
            pl.BlockSpec((1, 256), lambda i: (0, 0)),
            pl.BlockSpec(memory_space=pltpu.SMEM),
        ],
        out_specs=pl.BlockSpec((BT, 1), lambda i: (i, 0)),
        out_shape=jax.ShapeDtypeStruct((B, 1), jnp.float32),
    )(cnts, M_hi, M_lo, b1, W2, b2, W3, b3)


def kernel(x, side_to_move, emb, W1, b1, W2, b2, W3, b3):
    B = x.shape[0]
    x2d = x.astype(jnp.int32)
    S = 2
    h = B // S
    parts = [_sc_counts(x2d, side_to_move, s * h, h) for s in range(S)]
    M_hi, M_lo = _tc_fold(emb, W1)
    outs = [_tc_mlp(p, M_hi, M_lo, b1.reshape(1, D1), W2,
                    b2.reshape(1, 256), W3, b3.reshape(1, 1))
            for p in parts]
    return jnp.concatenate(outs, axis=0)


# speedup vs baseline: 1.8784x; 1.0265x over previous
import functools

import jax
import jax.numpy as jnp
from jax import lax
from jax.experimental import pallas as pl
from jax.experimental.pallas import tpu as pltpu
from jax.experimental.pallas import tpu_sc as plsc

VOCAB = 832
EMB_DIM = 1024
D1 = 512
W_CNT = 896
NC = 2
NS = 16
NW = NC * NS
LANES = 16


def _sc_counts(x2d, side, base, rows):
    b_per_w = rows // NW
    CH = LANES
    n_ch = b_per_w // CH

    mesh = plsc.VectorSubcoreMesh(core_axis_name="c", subcore_axis_name="s")

    @functools.partial(
        pl.kernel,
        out_type=jax.ShapeDtypeStruct((rows, W_CNT), jnp.float32),
        mesh=mesh,
        compiler_params=pltpu.CompilerParams(needs_layout_passes=False),
        scratch_types=[
            pltpu.VMEM((b_per_w, 64), jnp.int32),
            pltpu.VMEM((b_per_w,), jnp.float32),
            pltpu.VMEM((CH, W_CNT), jnp.float32),
            pltpu.VMEM((CH, W_CNT), jnp.float32),
            pltpu.VMEM((CH, W_CNT), jnp.float32),
            pltpu.VMEM((CH, W_CNT), jnp.float32),
            pltpu.SemaphoreType.DMA,
            pltpu.SemaphoreType.DMA,
        ],
    )
    def k(x_hbm, side_hbm, out_hbm, idx_all, side_all,
          cnt_v0, cnt_v1, cnt_v2, cnt_v3, isem, osem):
        cnt_b = (cnt_v0, cnt_v1, cnt_v2, cnt_v3)
        nbuf = len(cnt_b)
        wid = lax.axis_index("s") * NC + lax.axis_index("c")
        row0w = wid * b_per_w
        in0w = base + row0w
        lane = lax.iota(jnp.int32, 16)
        ones = jnp.ones((16,), jnp.float32)
        zeros = jnp.zeros((16,), jnp.float32)

        pltpu.async_copy(x_hbm.at[pl.ds(in0w, b_per_w)], idx_all, isem)
        pltpu.async_copy(side_hbm.at[pl.ds(in0w, b_per_w)], side_all, isem)
        for cb in range(nbuf):
            for r in range(CH):
                for i in range(W_CNT // 16):
                    cnt_b[cb][r, pl.ds(i * 16, 16)] = zeros
        pltpu.make_async_copy(
            x_hbm.at[pl.ds(in0w, b_per_w)], idx_all, isem).wait()
        pltpu.make_async_copy(
            side_hbm.at[pl.ds(in0w, b_per_w)], side_all, isem).wait()

        def chunk_quad(g, carry):
            for cb in range(nbuf):
                ch = nbuf * g + cb
                row0 = row0w + ch * CH
                rows = ch * CH + lane

                @pl.when(ch >= nbuf)
                def _drain():
                    pltpu.make_async_copy(
                        cnt_b[cb],
                        out_hbm.at[pl.ds(row0, CH)],
                        osem).wait()
                    prev_rows = rows - nbuf * CH
                    for sq in range(64):
                        iv = plsc.load_gather(
                            idx_all,
                            [prev_rows, jnp.full((16,), sq, jnp.int32)])
                        plsc.store_scatter(cnt_b[cb], [lane, iv], zeros)

                for sq in range(64):
                    iv = plsc.load_gather(
                        idx_all, [rows, jnp.full((16,), sq, jnp.int32)])
                    plsc.addupdate_scatter(cnt_b[cb], [lane, iv], ones)
                plsc.store_scatter(cnt_b[cb],
                                   [lane, jnp.full((16,), VOCAB, jnp.int32)],
                                   side_all[pl.ds(ch * CH, CH)])
                pltpu.async_copy(
                    cnt_b[cb],
                    out_hbm.at[pl.ds(row0, CH)], osem)
            return carry

        lax.fori_loop(0, n_ch // nbuf, chunk_quad, 0)

        for cb in range(nbuf):
            pltpu.make_async_copy(
                cnt_b[cb], out_hbm.at[pl.ds(row0w, CH)], osem).wait()

    return k(x2d, side)


def _tc_fold(emb, W1):
    def body(emb_ref, w1_ref, hi_ref, lo_ref):
        w1a = w1_ref[:, pl.ds(0, EMB_DIM)]
        m = lax.dot_general(
            emb_ref[...], w1a, (((1,), (1,)), ((), ())),
            preferred_element_type=jnp.float32,
            precision=lax.Precision.HIGHEST)
        w1s = jnp.reshape(w1_ref[:, pl.ds(EMB_DIM, 1)], (1, D1))
        pad = jnp.zeros((W_CNT - VOCAB - 1, D1), jnp.float32)
        m = jnp.concatenate([m, w1s, pad], axis=0)
        hi = m.astype(jnp.bfloat16)
        hi_ref[...] = hi
        lo_ref[...] = (m - hi.astype(jnp.float32)).astype(jnp.bfloat16)

    return pl.pallas_call(
        body,
        out_shape=[jax.ShapeDtypeStruct((W_CNT, D1), jnp.bfloat16),
                   jax.ShapeDtypeStruct((W_CNT, D1), jnp.bfloat16)],
    )(emb, W1)


def _tc_mlp(cnts, M_hi, M_lo, b1, W2, b2, W3, b3, out_total, row_off,
            out_prev=None):
    rows = cnts.shape[0]
    BT = 1024
    nb = rows // BT
    off = row_off // BT

    def body(c_ref, mh_ref, ml_ref, b1_ref, w2_ref, b2_ref, w3_ref, b3_ref,
             *rest):
        o_ref = rest[-1]
        cb = c_ref[...].astype(jnp.bfloat16)
        g = (jnp.dot(cb, mh_ref[...], preferred_element_type=jnp.float32)
             + jnp.dot(cb, ml_ref[...], preferred_element_type=jnp.float32))
        h1 = jnp.maximum(g + b1_ref[...], 0.0)
        h2 = lax.dot_general(h1, w2_ref[...], (((1,), (1,)), ((), ())),
                             preferred_element_type=jnp.float32)
        h2 = jnp.maximum(h2 + b2_ref[...], 0.0)
        h3 = jnp.sum(h2 * w3_ref[...], axis=1, keepdims=True)
        o_ref[...] = h3 + b3_ref[0, 0]

    in_specs = [
        pl.BlockSpec((BT, W_CNT), lambda i: (i, 0)),
        pl.BlockSpec((W_CNT, D1), lambda i: (0, 0)),
        pl.BlockSpec((W_CNT, D1), lambda i: (0, 0)),
        pl.BlockSpec((1, D1), lambda i: (0, 0)),
        pl.BlockSpec((256, D1), lambda i: (0, 0)),
        pl.BlockSpec((1, 256), lambda i: (0, 0)),
        pl.BlockSpec((1, 256), lambda i: (0, 0)),
        pl.BlockSpec(memory_space=pltpu.SMEM),
    ]
    args = [cnts, M_hi, M_lo, b1, W2, b2, W3, b3]
    aliases = {}
    if out_prev is not None:
        in_specs.append(pl.BlockSpec(memory_space=pl.ANY))
        args.append(out_prev)
        aliases = {8: 0}
    return pl.pallas_call(
        body,
        grid=(nb,),
        in_specs=in_specs,
        out_specs=pl.BlockSpec((BT, 1), lambda i: (i + off, 0)),
        out_shape=jax.ShapeDtypeStruct((out_total, 1), jnp.float32),
        input_output_aliases=aliases,
    )(*args)


def kernel(x, side_to_move, emb, W1, b1, W2, b2, W3, b3):
    B = x.shape[0]
    x2d = x.astype(jnp.int32)
    sizes = (10 * B // 16, 6 * B // 16)
    bases = (0, sizes[0])
    parts = [_sc_counts(x2d, side_to_move, b, r)
             for b, r in zip(bases, sizes)]
    M_hi, M_lo = _tc_fold(emb, W1)
    out = None
    for p, b in zip(parts, bases):
        out = _tc_mlp(p, M_hi, M_lo, b1.reshape(1, D1), W2,
                      b2.reshape(1, 256), W3, b3.reshape(1, 1),
                      B, b, out_prev=out)
    return out


# speedup vs baseline: 1.8812x; 1.0015x over previous
import functools

import jax
import jax.numpy as jnp
from jax import lax
from jax.experimental import pallas as pl
from jax.experimental.pallas import tpu as pltpu
from jax.experimental.pallas import tpu_sc as plsc

VOCAB = 832
EMB_DIM = 1024
D1 = 512
W_CNT = 896
NC = 2
NS = 16
NW = NC * NS
LANES = 16


def _sc_counts(x2d, side, base, rows):
    b_per_w = rows // NW
    CH = LANES
    n_ch = b_per_w // CH
    assert rows % (NW * CH * 4) == 0, rows

    mesh = plsc.VectorSubcoreMesh(core_axis_name="c", subcore_axis_name="s")

    @functools.partial(
        pl.kernel,
        out_type=jax.ShapeDtypeStruct((rows, W_CNT), jnp.float32),
        mesh=mesh,
        compiler_params=pltpu.CompilerParams(needs_layout_passes=False),
        scratch_types=[
            pltpu.VMEM((b_per_w, 64), jnp.int32),
            pltpu.VMEM((b_per_w,), jnp.float32),
            pltpu.VMEM((CH, W_CNT), jnp.float32),
            pltpu.VMEM((CH, W_CNT), jnp.float32),
            pltpu.VMEM((CH, W_CNT), jnp.float32),
            pltpu.VMEM((CH, W_CNT), jnp.float32),
            pltpu.SemaphoreType.DMA,
            pltpu.SemaphoreType.DMA,
        ],
    )
    def k(x_hbm, side_hbm, out_hbm, idx_all, side_all,
          cnt_v0, cnt_v1, cnt_v2, cnt_v3, isem, osem):
        cnt_b = (cnt_v0, cnt_v1, cnt_v2, cnt_v3)
        nbuf = len(cnt_b)
        wid = lax.axis_index("s") * NC + lax.axis_index("c")
        row0w = wid * b_per_w
        in0w = base + row0w
        lane = lax.iota(jnp.int32, 16)
        ones = jnp.ones((16,), jnp.float32)
        zeros = jnp.zeros((16,), jnp.float32)

        pltpu.async_copy(x_hbm.at[pl.ds(in0w, b_per_w)], idx_all, isem)
        pltpu.async_copy(side_hbm.at[pl.ds(in0w, b_per_w)], side_all, isem)
        for cb in range(nbuf):
            for r in range(CH):
                for i in range(W_CNT // 16):
                    cnt_b[cb][r, pl.ds(i * 16, 16)] = zeros
        pltpu.make_async_copy(
            x_hbm.at[pl.ds(in0w, b_per_w)], idx_all, isem).wait()
        pltpu.make_async_copy(
            side_hbm.at[pl.ds(in0w, b_per_w)], side_all, isem).wait()

        def chunk_quad(g, carry):
            for cb in range(nbuf):
                ch = nbuf * g + cb
                row0 = row0w + ch * CH
                rows = ch * CH + lane

                @pl.when(ch >= nbuf)
                def _drain():
                    pltpu.make_async_copy(
                        cnt_b[cb],
                        out_hbm.at[pl.ds(row0, CH)],
                        osem).wait()
                    prev_rows = rows - nbuf * CH
                    for sq in range(64):
                        iv = plsc.load_gather(
                            idx_all,
                            [prev_rows, jnp.full((16,), sq, jnp.int32)])
                        plsc.store_scatter(cnt_b[cb], [lane, iv], zeros)

                for sq in range(64):
                    iv = plsc.load_gather(
                        idx_all, [rows, jnp.full((16,), sq, jnp.int32)])
                    plsc.addupdate_scatter(cnt_b[cb], [lane, iv], ones)
                plsc.store_scatter(cnt_b[cb],
                                   [lane, jnp.full((16,), VOCAB, jnp.int32)],
                                   side_all[pl.ds(ch * CH, CH)])
                pltpu.async_copy(
                    cnt_b[cb],
                    out_hbm.at[pl.ds(row0, CH)], osem)
            return carry

        lax.fori_loop(0, n_ch // nbuf, chunk_quad, 0)

        for cb in range(nbuf):
            pltpu.make_async_copy(
                cnt_b[cb], out_hbm.at[pl.ds(row0w, CH)], osem).wait()

    return k(x2d, side)


def _tc_fold(emb, W1):
    def body(emb_ref, w1_ref, hi_ref, lo_ref):
        w1a = w1_ref[:, pl.ds(0, EMB_DIM)]
        m = lax.dot_general(
            emb_ref[...], w1a, (((1,), (1,)), ((), ())),
            preferred_element_type=jnp.float32,
            precision=lax.Precision.HIGHEST)
        w1s = jnp.reshape(w1_ref[:, pl.ds(EMB_DIM, 1)], (1, D1))
        pad = jnp.zeros((W_CNT - VOCAB - 1, D1), jnp.float32)
        m = jnp.concatenate([m, w1s, pad], axis=0)
        hi = m.astype(jnp.bfloat16)
        hi_ref[...] = hi
        lo_ref[...] = (m - hi.astype(jnp.float32)).astype(jnp.bfloat16)

    return pl.pallas_call(
        body,
        out_shape=[jax.ShapeDtypeStruct((W_CNT, D1), jnp.bfloat16),
                   jax.ShapeDtypeStruct((W_CNT, D1), jnp.bfloat16)],
    )(emb, W1)


def _tc_mlp(cnts, M_hi, M_lo, b1, W2, b2, W3, b3, out_total, row_off,
            out_prev=None):
    rows = cnts.shape[0]
    BT = 1024
    nb = rows // BT
    off = row_off // BT

    def body(c_ref, mh_ref, ml_ref, b1_ref, w2_ref, b2_ref, w3_ref, b3_ref,
             *rest):
        o_ref = rest[-1]
        cb = c_ref[...].astype(jnp.bfloat16)
        g = (jnp.dot(cb, mh_ref[...], preferred_element_type=jnp.float32)
             + jnp.dot(cb, ml_ref[...], preferred_element_type=jnp.float32))
        h1 = jnp.maximum(g + b1_ref[...], 0.0)
        h2 = lax.dot_general(h1, w2_ref[...], (((1,), (1,)), ((), ())),
                             preferred_element_type=jnp.float32)
        h2 = jnp.maximum(h2 + b2_ref[...], 0.0)
        h3 = jnp.sum(h2 * w3_ref[...], axis=1, keepdims=True)
        o_ref[...] = h3 + b3_ref[0, 0]

    in_specs = [
        pl.BlockSpec((BT, W_CNT), lambda i: (i, 0)),
        pl.BlockSpec((W_CNT, D1), lambda i: (0, 0)),
        pl.BlockSpec((W_CNT, D1), lambda i: (0, 0)),
        pl.BlockSpec((1, D1), lambda i: (0, 0)),
        pl.BlockSpec((256, D1), lambda i: (0, 0)),
        pl.BlockSpec((1, 256), lambda i: (0, 0)),
        pl.BlockSpec((1, 256), lambda i: (0, 0)),
        pl.BlockSpec(memory_space=pltpu.SMEM),
    ]
    args = [cnts, M_hi, M_lo, b1, W2, b2, W3, b3]
    aliases = {}
    if out_prev is not None:
        in_specs.append(pl.BlockSpec(memory_space=pl.ANY))
        args.append(out_prev)
        aliases = {8: 0}
    return pl.pallas_call(
        body,
        grid=(nb,),
        in_specs=in_specs,
        out_specs=pl.BlockSpec((BT, 1), lambda i: (i + off, 0)),
        out_shape=jax.ShapeDtypeStruct((out_total, 1), jnp.float32),
        input_output_aliases=aliases,
    )(*args)


def kernel(x, side_to_move, emb, W1, b1, W2, b2, W3, b3):
    B = x.shape[0]
    x2d = x.astype(jnp.int32)
    sizes = (10 * B // 16, 6 * B // 16)
    bases = (0, sizes[0])
    parts = [_sc_counts(x2d, side_to_move, b, r)
             for b, r in zip(bases, sizes)]
    M_hi, M_lo = _tc_fold(emb, W1)
    out = None
    for p, b in zip(parts, bases):
        out = _tc_mlp(p, M_hi, M_lo, b1.reshape(1, D1), W2,
                      b2.reshape(1, 256), W3, b3.reshape(1, 1),
                      B, b, out_prev=out)
    return out
